# baseline (device time: 96991 ns/iter reference)
import jax
import jax.numpy as jnp
from jax import lax
from jax.experimental import pallas as pl
from jax.experimental.pallas import tpu as pltpu

N_DEV = 4


def kernel(A, B):
    A = A.astype(jnp.bfloat16)
    B = B.astype(jnp.bfloat16)
    m, _ = A.shape
    _, n = B.shape
    chunk = m // N_DEV

    def body(a_ref, b_ref, out_ref, comm_ref, send_sems, recv_sems):
        my = lax.axis_index("i")
        right = lax.rem(my + 1, N_DEV)

        def local_chunk(c):
            a = a_ref[pl.ds(c * chunk, chunk), :]
            return jnp.dot(a, b_ref[...], preferred_element_type=jnp.float32)

        c0 = lax.rem(my + N_DEV - 1, N_DEV)
        comm_ref[0, :, :] = local_chunk(c0).astype(comm_ref.dtype)

        for s in range(N_DEV - 1):
            rdma = pltpu.make_async_remote_copy(
                src_ref=comm_ref.at[s],
                dst_ref=comm_ref.at[s + 1],
                send_sem=send_sems.at[s],
                recv_sem=recv_sems.at[s],
                device_id=(right,),
                device_id_type=pl.DeviceIdType.MESH,
            )
            rdma.start()
            rdma.wait()
            c = lax.rem(my + 2 * N_DEV - 2 - s, N_DEV)
            partial = local_chunk(c) + comm_ref[s + 1, :, :].astype(jnp.float32)
            if s < N_DEV - 2:
                comm_ref[s + 1, :, :] = partial.astype(comm_ref.dtype)
            else:
                out_ref[...] = partial

    return pl.pallas_call(
        body,
        out_shape=jax.ShapeDtypeStruct((chunk, n), jnp.float32),
        in_specs=[
            pl.BlockSpec(memory_space=pltpu.VMEM),
            pl.BlockSpec(memory_space=pltpu.VMEM),
        ],
        out_specs=pl.BlockSpec(memory_space=pltpu.VMEM),
        scratch_shapes=[
            pltpu.VMEM((N_DEV, chunk, n), jnp.bfloat16),
            pltpu.SemaphoreType.DMA((N_DEV - 1,)),
            pltpu.SemaphoreType.DMA((N_DEV - 1,)),
        ],
    )(A, B)


# device time: 64353 ns/iter; 1.5072x vs baseline; 1.5072x over previous
import jax
import jax.numpy as jnp
from jax import lax
from jax.experimental import pallas as pl
from jax.experimental.pallas import tpu as pltpu

N_DEV = 4


def kernel(A, B):
    A = A.astype(jnp.bfloat16)
    B = B.astype(jnp.bfloat16)
    m, _ = A.shape
    _, n = B.shape
    chunk = m // N_DEV
    half = n // 2

    def body(a_ref, b_ref, out_ref, comm_r, comm_l, send_r, recv_r,
             send_l, recv_l):
        my = lax.axis_index("i")
        right = lax.rem(my + 1, N_DEV)
        left = lax.rem(my + N_DEV - 1, N_DEV)

        def partial_r(c):
            a = a_ref[pl.ds(c * chunk, chunk), :]
            return jnp.dot(a, b_ref[:, :half],
                           preferred_element_type=jnp.float32)

        def partial_l(c):
            a = a_ref[pl.ds(c * chunk, chunk), :]
            return jnp.dot(a, b_ref[:, half:],
                           preferred_element_type=jnp.float32)

        comm_r[0, :, :] = partial_r(lax.rem(my + N_DEV - 1, N_DEV)).astype(
            comm_r.dtype)
        comm_l[0, :, :] = partial_l(lax.rem(my + 1, N_DEV)).astype(
            comm_l.dtype)

        for s in range(N_DEV - 1):
            rdma_r = pltpu.make_async_remote_copy(
                src_ref=comm_r.at[s],
                dst_ref=comm_r.at[s + 1],
                send_sem=send_r.at[s],
                recv_sem=recv_r.at[s],
                device_id=(right,),
                device_id_type=pl.DeviceIdType.MESH,
            )
            rdma_l = pltpu.make_async_remote_copy(
                src_ref=comm_l.at[s],
                dst_ref=comm_l.at[s + 1],
                send_sem=send_l.at[s],
                recv_sem=recv_l.at[s],
                device_id=(left,),
                device_id_type=pl.DeviceIdType.MESH,
            )
            rdma_r.start()
            rdma_l.start()
            p_r = partial_r(lax.rem(my + 2 * N_DEV - 2 - s, N_DEV))
            p_l = partial_l(lax.rem(my + 2 + s, N_DEV))
            rdma_r.wait()
            if s < N_DEV - 2:
                comm_r[s + 1, :, :] = (
                    p_r + comm_r[s + 1, :, :].astype(jnp.float32)
                ).astype(comm_r.dtype)
            else:
                out_ref[:, :half] = p_r + comm_r[s + 1, :, :].astype(
                    jnp.float32)
            rdma_l.wait()
            if s < N_DEV - 2:
                comm_l[s + 1, :, :] = (
                    p_l + comm_l[s + 1, :, :].astype(jnp.float32)
                ).astype(comm_l.dtype)
            else:
                out_ref[:, half:] = p_l + comm_l[s + 1, :, :].astype(
                    jnp.float32)

    return pl.pallas_call(
        body,
        out_shape=jax.ShapeDtypeStruct((chunk, n), jnp.float32),
        in_specs=[
            pl.BlockSpec(memory_space=pltpu.VMEM),
            pl.BlockSpec(memory_space=pltpu.VMEM),
        ],
        out_specs=pl.BlockSpec(memory_space=pltpu.VMEM),
        scratch_shapes=[
            pltpu.VMEM((N_DEV, chunk, half), jnp.bfloat16),
            pltpu.VMEM((N_DEV, chunk, half), jnp.bfloat16),
            pltpu.SemaphoreType.DMA((N_DEV - 1,)),
            pltpu.SemaphoreType.DMA((N_DEV - 1,)),
            pltpu.SemaphoreType.DMA((N_DEV - 1,)),
            pltpu.SemaphoreType.DMA((N_DEV - 1,)),
        ],
    )(A, B)


# device time: 52847 ns/iter; 1.8353x vs baseline; 1.2177x over previous
import jax
import jax.numpy as jnp
from jax import lax
from jax.experimental import pallas as pl
from jax.experimental.pallas import tpu as pltpu

N_DEV = 4
RAILS = 2


def kernel(A, B):
    m, _ = A.shape
    _, n = B.shape
    chunk = m // N_DEV
    half = n // 2
    rail = half // RAILS

    def body(a_ref, b_ref, out_ref, comm_r, comm_l, b_bf,
             send_r, recv_r, send_l, recv_l):
        my = lax.axis_index("i")
        right = lax.rem(my + 1, N_DEV)
        left = lax.rem(my + N_DEV - 1, N_DEV)

        b_bf[...] = b_ref[...].astype(jnp.bfloat16)

        def a_slice(c):
            return a_ref[pl.ds(c * chunk, chunk), :].astype(jnp.bfloat16)

        def dot_half(c, lo):
            return jax.lax.dot_general(
                a_slice(c), b_bf[:, lo:lo + half],
                (((1,), (0,)), ((), ())),
                preferred_element_type=jnp.float32,
            )

        def make(comm, sems_s, sems_r, r, s, dst):
            return pltpu.make_async_remote_copy(
                src_ref=comm.at[r, s],
                dst_ref=comm.at[r, s + 1],
                send_sem=sems_s.at[r, s],
                recv_sem=sems_r.at[r, s],
                device_id=(dst,),
                device_id_type=pl.DeviceIdType.MESH,
            )

        p = dot_half(lax.rem(my + N_DEV - 1, N_DEV), 0)
        pend_r = []
        for r in range(RAILS):
            comm_r[r, 0, :, :] = p[:, r * rail:(r + 1) * rail].astype(
                comm_r.dtype)
            d = make(comm_r, send_r, recv_r, r, 0, right)
            d.start()
            pend_r.append(d)
        p = dot_half(lax.rem(my + 1, N_DEV), half)
        pend_l = []
        for r in range(RAILS):
            comm_l[r, 0, :, :] = p[:, r * rail:(r + 1) * rail].astype(
                comm_l.dtype)
            d = make(comm_l, send_l, recv_l, r, 0, left)
            d.start()
            pend_l.append(d)

        for s in range(N_DEV - 1):
            p_r = dot_half(lax.rem(my + 2 * N_DEV - 2 - s, N_DEV), 0)
            p_l = dot_half(lax.rem(my + 2 + s, N_DEV), half)
            for r in range(RAILS):
                for comm, pend, sems_s, sems_r, dst, p, lo in (
                    (comm_r, pend_r, send_r, recv_r, right, p_r, 0),
                    (comm_l, pend_l, send_l, recv_l, left, p_l, half),
                ):
                    pend[r].wait()
                    acc = p[:, r * rail:(r + 1) * rail]
                    if s < N_DEV - 2:
                        comm[r, s + 1, :, :] = (
                            acc + comm[r, s + 1, :, :].astype(jnp.float32)
                        ).astype(comm.dtype)
                        d = make(comm, sems_s, sems_r, r, s + 1, dst)
                        d.start()
                        pend[r] = d
                    else:
                        c0 = lo + r * rail
                        out_ref[:, c0:c0 + rail] = (
                            acc + comm[r, s + 1, :, :].astype(jnp.float32))

    return pl.pallas_call(
        body,
        out_shape=jax.ShapeDtypeStruct((chunk, n), jnp.float32),
        in_specs=[
            pl.BlockSpec(memory_space=pltpu.VMEM),
            pl.BlockSpec(memory_space=pltpu.VMEM),
        ],
        out_specs=pl.BlockSpec(memory_space=pltpu.VMEM),
        scratch_shapes=[
            pltpu.VMEM((RAILS, N_DEV, chunk, rail), jnp.bfloat16),
            pltpu.VMEM((RAILS, N_DEV, chunk, rail), jnp.bfloat16),
            pltpu.VMEM((A.shape[1], n), jnp.bfloat16),
            pltpu.SemaphoreType.DMA((RAILS, N_DEV - 1)),
            pltpu.SemaphoreType.DMA((RAILS, N_DEV - 1)),
            pltpu.SemaphoreType.DMA((RAILS, N_DEV - 1)),
            pltpu.SemaphoreType.DMA((RAILS, N_DEV - 1)),
        ],
    )(A, B)


# device time: 49409 ns/iter; 1.9630x vs baseline; 1.0696x over previous
import jax
import jax.numpy as jnp
from jax import lax
from jax.experimental import pallas as pl
from jax.experimental.pallas import tpu as pltpu

N_DEV = 4
RAILS = 2


def kernel(A, B):
    m, _ = A.shape
    _, n = B.shape
    chunk = m // N_DEV
    half = n // 2
    rail = half // RAILS

    def body(a_ref, b_ref, out_ref, comm_r, comm_l, b_bf,
             send_r, recv_r, send_l, recv_l):
        my = lax.axis_index("i")
        right = lax.rem(my + 1, N_DEV)
        left = lax.rem(my + N_DEV - 1, N_DEV)

        def a_slice(c):
            return a_ref[pl.ds(c * chunk, chunk), :].astype(jnp.bfloat16)

        def dot_half(c, lo):
            return jax.lax.dot_general(
                a_slice(c), b_bf[:, lo:lo + half],
                (((1,), (0,)), ((), ())),
                preferred_element_type=jnp.float32,
            )

        def make(comm, sems_s, sems_r, r, s, dst):
            return pltpu.make_async_remote_copy(
                src_ref=comm.at[r, s],
                dst_ref=comm.at[r, s + 1],
                send_sem=sems_s.at[r, s],
                recv_sem=sems_r.at[r, s],
                device_id=(dst,),
                device_id_type=pl.DeviceIdType.MESH,
            )

        c_r0 = lax.rem(my + N_DEV - 1, N_DEV)
        c_l0 = lax.rem(my + 1, N_DEV)
        a_r = a_slice(c_r0)
        a_l = a_slice(c_l0)
        pend_r = [None] * RAILS
        pend_l = [None] * RAILS
        first = True
        for r in range(RAILS):
            for comm, pend, sems_s, sems_r, dst, a_bf, lo in (
                (comm_r, pend_r, send_r, recv_r, right, a_r, 0),
                (comm_l, pend_l, send_l, recv_l, left, a_l, half),
            ):
                c0 = lo + r * rail
                b_bf[:, c0:c0 + rail] = b_ref[:, c0:c0 + rail].astype(
                    jnp.bfloat16)
                p = jax.lax.dot_general(
                    a_bf, b_bf[:, c0:c0 + rail],
                    (((1,), (0,)), ((), ())),
                    preferred_element_type=jnp.float32,
                )
                comm[r, 0, :, :] = p.astype(comm.dtype)
                if first:
                    barrier = pltpu.get_barrier_semaphore()
                    for nbr in (left, right):
                        pl.semaphore_signal(
                            barrier, inc=1, device_id=(nbr,),
                            device_id_type=pl.DeviceIdType.MESH)
                    pl.semaphore_wait(barrier, 2)
                    first = False
                d = make(comm, sems_s, sems_r, r, 0, dst)
                d.start()
                pend[r] = d

        for s in range(N_DEV - 1):
            p_r = dot_half(lax.rem(my + 2 * N_DEV - 2 - s, N_DEV), 0)
            p_l = dot_half(lax.rem(my + 2 + s, N_DEV), half)
            for r in range(RAILS):
                for comm, pend, sems_s, sems_r, dst, p, lo in (
                    (comm_r, pend_r, send_r, recv_r, right, p_r, 0),
                    (comm_l, pend_l, send_l, recv_l, left, p_l, half),
                ):
                    pend[r].wait()
                    acc = p[:, r * rail:(r + 1) * rail]
                    if s < N_DEV - 2:
                        comm[r, s + 1, :, :] = (
                            acc + comm[r, s + 1, :, :].astype(jnp.float32)
                        ).astype(comm.dtype)
                        d = make(comm, sems_s, sems_r, r, s + 1, dst)
                        d.start()
                        pend[r] = d
                    else:
                        c0 = lo + r * rail
                        out_ref[:, c0:c0 + rail] = (
                            acc + comm[r, s + 1, :, :].astype(jnp.float32))

    return pl.pallas_call(
        body,
        out_shape=jax.ShapeDtypeStruct((chunk, n), jnp.float32),
        in_specs=[
            pl.BlockSpec(memory_space=pltpu.VMEM),
            pl.BlockSpec(memory_space=pltpu.VMEM),
        ],
        out_specs=pl.BlockSpec(memory_space=pltpu.VMEM),
        scratch_shapes=[
            pltpu.VMEM((RAILS, N_DEV, chunk, rail), jnp.bfloat16),
            pltpu.VMEM((RAILS, N_DEV, chunk, rail), jnp.bfloat16),
            pltpu.VMEM((A.shape[1], n), jnp.bfloat16),
            pltpu.SemaphoreType.DMA((RAILS, N_DEV - 1)),
            pltpu.SemaphoreType.DMA((RAILS, N_DEV - 1)),
            pltpu.SemaphoreType.DMA((RAILS, N_DEV - 1)),
            pltpu.SemaphoreType.DMA((RAILS, N_DEV - 1)),
        ],
        compiler_params=pltpu.CompilerParams(collective_id=0),
    )(A, B)
